# baseline (speedup 1.0000x reference)
import jax
import jax.numpy as jnp
from jax.experimental import pallas as pl
from jax.experimental.pallas import tpu as pltpu
from jax.experimental.pallas import tpu_sc as plsc

D_MODEL = 1024
NUM_TILES = 32
CHUNK = 16
NBUF = 4


def _sc_gather(table, idx2d, n):
    per_tile = n // NUM_TILES
    nchunk = per_tile // CHUNK
    mesh = plsc.VectorSubcoreMesh(core_axis_name="c", subcore_axis_name="s")

    @pl.kernel(
        out_type=jax.ShapeDtypeStruct((n, D_MODEL), table.dtype),
        mesh=mesh,
        scratch_types=[
            pltpu.VMEM((1, per_tile), jnp.int32),
        ]
        + [pltpu.VMEM((CHUNK, D_MODEL), jnp.float32) for _ in range(NBUF)]
        + [pltpu.SemaphoreType.DMA for _ in range(2 * NBUF)],
    )
    def gather_kernel(table_hbm, idx_hbm, out_hbm, idx_v, *bufs_and_sems):
        bufs = bufs_and_sems[:NBUF]
        gsems = bufs_and_sems[NBUF : 2 * NBUF]
        wsems = bufs_and_sems[2 * NBUF : 3 * NBUF]

        c = jax.lax.axis_index("c")
        s = jax.lax.axis_index("s")
        t = c * 16 + s
        pltpu.async_copy(idx_hbm.at[pl.ds(t, 1), :], idx_v, gsems[0]).wait()
        row0 = t * per_tile

        def gather_copy(chunk, b):
            rows = idx_v.at[0, pl.ds(chunk * CHUNK, CHUNK)]
            return pltpu.make_async_copy(table_hbm.at[rows], bufs[b], gsems[b])

        def write_copy(chunk, b):
            dst = out_hbm.at[pl.ds(row0 + chunk * CHUNK, CHUNK), :]
            return pltpu.make_async_copy(bufs[b], dst, wsems[b])

        for b in range(NBUF):
            gather_copy(b, b).start()

        @pl.loop(0, nchunk - NBUF, step=NBUF)
        def _(j):
            for b in range(NBUF):
                gather_copy(j + b, b).wait()
                write_copy(j + b, b).start()
            for b in range(NBUF):
                write_copy(j + b, b).wait()
                gather_copy(j + NBUF + b, b).start()

        for b in range(NBUF):
            gather_copy(nchunk - NBUF + b, b).wait()
            write_copy(nchunk - NBUF + b, b).start()
        for b in range(NBUF):
            write_copy(nchunk - NBUF + b, b).wait()

    return gather_kernel(table, idx2d)


def kernel(mask_bits, table):
    b, s = mask_bits.shape
    n = b * s
    idx = mask_bits.astype(jnp.int32).reshape(NUM_TILES, n // NUM_TILES)
    reps = 32
    table_rep = jnp.tile(table, (NUM_TILES * reps, 1))
    tile_off = jnp.arange(NUM_TILES, dtype=jnp.int32)[:, None] * reps
    pos_off = jnp.arange(n // NUM_TILES, dtype=jnp.int32)[None, :] % reps
    idx = idx + 2 * (tile_off + pos_off)
    out = _sc_gather(table_rep, idx, n)
    return out.reshape(b, s, D_MODEL)


# speedup vs baseline: 1.0331x; 1.0331x over previous
import jax
import jax.numpy as jnp
from jax.experimental import pallas as pl
from jax.experimental.pallas import tpu as pltpu
from jax.experimental.pallas import tpu_sc as plsc

D_MODEL = 1024
NUM_TILES = 32
CHUNK = 32
NBUF = 2


def _sc_gather(table, idx2d, n):
    per_tile = n // NUM_TILES
    nchunk = per_tile // CHUNK
    mesh = plsc.VectorSubcoreMesh(core_axis_name="c", subcore_axis_name="s")

    @pl.kernel(
        out_type=jax.ShapeDtypeStruct((n, D_MODEL), table.dtype),
        mesh=mesh,
        scratch_types=[
            pltpu.VMEM((1, per_tile), jnp.int32),
        ]
        + [pltpu.VMEM((CHUNK, D_MODEL), jnp.float32) for _ in range(NBUF)]
        + [pltpu.SemaphoreType.DMA for _ in range(2 * NBUF)],
    )
    def gather_kernel(table_hbm, idx_hbm, out_hbm, idx_v, *bufs_and_sems):
        bufs = bufs_and_sems[:NBUF]
        gsems = bufs_and_sems[NBUF : 2 * NBUF]
        wsems = bufs_and_sems[2 * NBUF : 3 * NBUF]

        c = jax.lax.axis_index("c")
        s = jax.lax.axis_index("s")
        t = c * 16 + s
        pltpu.async_copy(idx_hbm.at[pl.ds(t, 1), :], idx_v, gsems[0]).wait()
        row0 = t * per_tile

        def gather_copy(chunk, b):
            rows = idx_v.at[0, pl.ds(chunk * CHUNK, CHUNK)]
            return pltpu.make_async_copy(table_hbm.at[rows], bufs[b], gsems[b])

        def write_copy(chunk, b):
            dst = out_hbm.at[pl.ds(row0 + chunk * CHUNK, CHUNK), :]
            return pltpu.make_async_copy(bufs[b], dst, wsems[b])

        for b in range(NBUF):
            gather_copy(b, b).start()

        @pl.loop(0, nchunk - NBUF, step=NBUF)
        def _(j):
            for b in range(NBUF):
                gather_copy(j + b, b).wait()
                write_copy(j + b, b).start()
            for b in range(NBUF):
                write_copy(j + b, b).wait()
                gather_copy(j + NBUF + b, b).start()

        for b in range(NBUF):
            gather_copy(nchunk - NBUF + b, b).wait()
            write_copy(nchunk - NBUF + b, b).start()
        for b in range(NBUF):
            write_copy(nchunk - NBUF + b, b).wait()

    return gather_kernel(table, idx2d)


def kernel(mask_bits, table):
    b, s = mask_bits.shape
    n = b * s
    idx = mask_bits.astype(jnp.int32).reshape(NUM_TILES, n // NUM_TILES)
    reps = 16
    table_rep = jnp.tile(table, (NUM_TILES * reps, 1))
    tile_off = jnp.arange(NUM_TILES, dtype=jnp.int32)[:, None] * reps
    pos_off = jnp.arange(n // NUM_TILES, dtype=jnp.int32)[None, :] % reps
    idx = idx + 2 * (tile_off + pos_off)
    out = _sc_gather(table_rep, idx, n)
    return out.reshape(b, s, D_MODEL)


# speedup vs baseline: 1.0622x; 1.0281x over previous
import jax
import jax.numpy as jnp
from jax.experimental import pallas as pl
from jax.experimental.pallas import tpu as pltpu
from jax.experimental.pallas import tpu_sc as plsc

D_MODEL = 1024
NUM_TILES = 32
CHUNK = 8
NBUF = 8


def _sc_gather(table, idx2d, n):
    per_tile = n // NUM_TILES
    nchunk = per_tile // CHUNK
    mesh = plsc.VectorSubcoreMesh(core_axis_name="c", subcore_axis_name="s")

    @pl.kernel(
        out_type=jax.ShapeDtypeStruct((n, D_MODEL), table.dtype),
        mesh=mesh,
        scratch_types=[
            pltpu.VMEM((1, per_tile), jnp.int32),
        ]
        + [pltpu.VMEM((CHUNK, D_MODEL), jnp.float32) for _ in range(NBUF)]
        + [pltpu.SemaphoreType.DMA for _ in range(2 * NBUF)],
    )
    def gather_kernel(table_hbm, idx_hbm, out_hbm, idx_v, *bufs_and_sems):
        bufs = bufs_and_sems[:NBUF]
        gsems = bufs_and_sems[NBUF : 2 * NBUF]
        wsems = bufs_and_sems[2 * NBUF : 3 * NBUF]

        c = jax.lax.axis_index("c")
        s = jax.lax.axis_index("s")
        t = c * 16 + s
        pltpu.async_copy(idx_hbm.at[pl.ds(t, 1), :], idx_v, gsems[0]).wait()
        row0 = t * per_tile

        def gather_copy(chunk, b):
            rows = idx_v.at[0, pl.ds(chunk * CHUNK, CHUNK)]
            return pltpu.make_async_copy(table_hbm.at[rows], bufs[b], gsems[b])

        def write_copy(chunk, b):
            dst = out_hbm.at[pl.ds(row0 + chunk * CHUNK, CHUNK), :]
            return pltpu.make_async_copy(bufs[b], dst, wsems[b])

        for b in range(NBUF):
            gather_copy(b, b).start()

        @pl.loop(0, nchunk - NBUF, step=NBUF)
        def _(j):
            for b in range(NBUF):
                gather_copy(j + b, b).wait()
                write_copy(j + b, b).start()
            for b in range(NBUF):
                write_copy(j + b, b).wait()
                gather_copy(j + NBUF + b, b).start()

        for b in range(NBUF):
            gather_copy(nchunk - NBUF + b, b).wait()
            write_copy(nchunk - NBUF + b, b).start()
        for b in range(NBUF):
            write_copy(nchunk - NBUF + b, b).wait()

    return gather_kernel(table, idx2d)


def kernel(mask_bits, table):
    b, s = mask_bits.shape
    n = b * s
    idx = mask_bits.astype(jnp.int32).reshape(NUM_TILES, n // NUM_TILES)
    reps = 16
    table_rep = jnp.tile(table, (NUM_TILES * reps, 1))
    tile_off = jnp.arange(NUM_TILES, dtype=jnp.int32)[:, None] * reps
    pos_off = jnp.arange(n // NUM_TILES, dtype=jnp.int32)[None, :] % reps
    idx = idx + 2 * (tile_off + pos_off)
    out = _sc_gather(table_rep, idx, n)
    return out.reshape(b, s, D_MODEL)
